# baseline (speedup 1.0000x reference)
import functools

import jax
import jax.numpy as jnp
from jax import lax
from jax.experimental import pallas as pl
from jax.experimental.pallas import tpu as pltpu
from jax.experimental.pallas import tpu_sc as plsc

EMB = 64
BATCH = 16384
NC = 2
NS = 16
NW = NC * NS
BPW = BATCH // NW
L = 16
VOCAB = 1000000
FULL_TILES = VOCAB // 128
TAIL_C0 = FULL_TILES * 128
SLAB_TILES = 4
SLAB_COLS = SLAB_TILES * 128
SLAB_WORDS = EMB * SLAB_COLS
GROWS = BATCH + 2 * NW
NSLOTS = 32
SENTINEL = 1 << 30


SUPER = 8


def _gather_body(idx_t_hbm, idx_c_hbm, emb_t_hbm, emb_c_hbm,
                 g_t_hbm, g_c_hbm,
                 idx_all, pos_list, sub_pos, slab, rstage,
                 tmp_val, tmp_pos, sem_slab, sem_row):
    wid = lax.axis_index("s") * NC + lax.axis_index("c")
    lanes = lax.iota(jnp.int32, L)

    lo_tile = (wid * FULL_TILES) // NW
    hi_tile = ((wid + 1) * FULL_TILES) // NW
    nslabs = (hi_tile - lo_tile + SLAB_TILES - 1) // SLAB_TILES
    nsuper = (nslabs + SUPER - 1) // SUPER
    lo_col = lo_tile * 128
    hi_col = jnp.where(wid == NW - 1, VOCAB, hi_tile * 128)

    featv = [lanes + q * L for q in range(EMB // L)]

    def slab_c0(s):
        st = jnp.minimum(lo_tile + s * SLAB_TILES, hi_tile - SLAB_TILES)
        return st * 128

    def fire_slab(s, emb_hbm):
        halfc = lax.rem(s, 2) * SLAB_COLS
        c0 = slab_c0(s)
        for hr in range(4):
            pltpu.async_copy(
                emb_hbm.at[pl.ds(hr * 16, 16), pl.ds(c0, SLAB_COLS)],
                slab.at[pl.ds(hr * 16, 16), pl.ds(halfc, SLAB_COLS)],
                sem_slab)

    def drain_slab():
        pltpu.make_async_copy(
            emb_t_hbm.at[pl.ds(0, EMB), pl.ds(0, SLAB_COLS)],
            slab.at[pl.ds(0, EMB), pl.ds(0, SLAB_COLS)], sem_slab).wait()

    def drain_row():
        pltpu.make_async_copy(
            emb_t_hbm.at[0, pl.ds(0, 128)],
            rstage.at[pl.ds(0, 128)], sem_row).wait()

    def extract(list_ref, listlen, c0, c1, half, hh0, g_hbm):
        nv = (listlen + L - 1) // L

        def scan_one(kk, hh):
            posv = list_ref[pl.ds(kk * L, L)]
            valv = plsc.load_gather(idx_all, [posv])
            m = (valv >= c0) & (valv < c1)
            n = plsc.all_reduce_population_count(m)[0]
            plsc.store_compressed(tmp_val.at[pl.ds(0, L)], valv, mask=m)
            plsc.store_compressed(tmp_pos.at[pl.ds(0, L)], posv, mask=m)

            def hit_one(h, hh):
                cv = tmp_val[pl.ds(h, L)][0]
                pos = tmp_pos[pl.ds(h, L)][0]
                cl = half + (cv - c0)
                clv = jnp.full((L,), 0, jnp.int32) + cl
                slot = lax.rem(hh, NSLOTS)
                drain_row()
                for q in range(EMB // L):
                    colv = plsc.load_gather(slab, [featv[q], clv])
                    rstage[pl.ds(slot * 128 + q * L, L)] = colv
                pltpu.async_copy(rstage.at[pl.ds(slot * 128, 128)],
                                 g_hbm.at[pos], sem_row)
                return hh + 1

            return lax.fori_loop(0, n, hit_one, hh)

        return lax.fori_loop(0, nv, scan_one, hh0)

    for (idx_hbm, emb_hbm, g_hbm, padrow) in (
            (idx_t_hbm, emb_t_hbm, g_t_hbm, BATCH),
            (idx_c_hbm, emb_c_hbm, g_c_hbm, BATCH + NW)):
        pltpu.sync_copy(idx_hbm, idx_all.at[pl.ds(0, BATCH)])
        idx_all[pl.ds(BATCH, L)] = jnp.full((L,), SENTINEL, jnp.int32)

        def build_one(k, count):
            v = idx_all[pl.ds(k * L, L)]
            m = (v >= lo_col) & (v < hi_col)
            plsc.store_compressed(
                pos_list.at[pl.ds(count, L)], lanes + k * L, mask=m)
            return count + plsc.all_reduce_population_count(m)[0]

        count = lax.fori_loop(0, BATCH // L, build_one, 0)
        pos_list[pl.ds(count, L)] = jnp.full((L,), BATCH, jnp.int32)

        for d in range(NSLOTS):
            pltpu.async_copy(rstage.at[pl.ds((d % NSLOTS) * 128, 128)],
                             g_hbm.at[padrow + d % NW], sem_row)

        fire_slab(0, emb_hbm)

        def super_step(sp, hh):
            sc0 = slab_c0(sp * SUPER)
            sc1 = sc0 + SUPER * SLAB_COLS

            def filt(kk, scount):
                posv = pos_list[pl.ds(kk * L, L)]
                valv = plsc.load_gather(idx_all, [posv])
                m = (valv >= sc0) & (valv < sc1)
                plsc.store_compressed(
                    sub_pos.at[pl.ds(scount, L)], posv, mask=m)
                return scount + plsc.all_reduce_population_count(m)[0]

            scount = lax.fori_loop(0, (count + L - 1) // L, filt, 0)
            sub_pos[pl.ds(scount, L)] = jnp.full((L,), BATCH, jnp.int32)

            def slab_step(k, hh):
                sg = sp * SUPER + k
                fire_slab(sg + 1, emb_hbm)
                drain_slab()
                c0 = slab_c0(sg)
                hh = extract(sub_pos, scount, c0, c0 + SLAB_COLS,
                             lax.rem(sg, 2) * SLAB_COLS, hh, g_hbm)
                return hh

            return lax.fori_loop(0, SUPER, slab_step, hh)

        hh = lax.fori_loop(0, nsuper, super_step, 0)
        drain_slab()

        for f in range(EMB):
            pltpu.async_copy(
                emb_hbm.at[f, pl.ds(TAIL_C0, 64)],
                slab.at[f, pl.ds(0, 64)], sem_slab)
        pltpu.make_async_copy(
            emb_t_hbm.at[pl.ds(0, 8), pl.ds(0, SLAB_COLS)],
            slab.at[pl.ds(0, 8), pl.ds(0, SLAB_COLS)], sem_slab).wait()
        hh = extract(pos_list, count, TAIL_C0, TAIL_C0 + 64, 0, hh, g_hbm)

        def drain_rest(_, x):
            drain_row()
            return x

        lax.fori_loop(0, NSLOTS, drain_rest, 0)


def _dot_body(g_t_hbm, g_c_hbm, wb_hbm, out_hbm,
              t_loc, c_loc, out_v, wb_v, scr, sem):
    wid = lax.axis_index("s") * NC + lax.axis_index("c")
    base = wid * BPW
    lanes = lax.iota(jnp.int32, L)

    pltpu.sync_copy(wb_hbm, wb_v)

    HALF = 256
    for h in range(BPW // HALF):
        pltpu.sync_copy(
            g_t_hbm.at[pl.ds(base + h * HALF, HALF), pl.ds(0, 128)], t_loc)
        pltpu.sync_copy(
            g_c_hbm.at[pl.ds(base + h * HALF, HALF), pl.ds(0, 128)], c_loc)

        def group(g, _):
            for i in range(L):
                row = g * L + i
                s = t_loc[row, pl.ds(0, L)] * c_loc[row, pl.ds(0, L)]
                for q in range(1, EMB // L):
                    s = s + (t_loc[row, pl.ds(q * L, L)]
                             * c_loc[row, pl.ds(q * L, L)])
                scr[pl.ds(i * L, L)] = s
            acc = plsc.load_gather(scr, [lanes * L])
            for j in range(1, L):
                acc = acc + plsc.load_gather(scr, [lanes * L + j])
            out_v[pl.ds(h * HALF + g * L, L)] = acc
            return _

        lax.fori_loop(0, HALF // L, group, None)

    w = wb_v[0, pl.ds(0, L)]
    bb = wb_v[1, pl.ds(0, L)]
    for j in range(BPW // L):
        v = out_v[pl.ds(j * L, L)]
        z = v * w + bb
        out_v[pl.ds(j * L, L)] = 1.0 / (1.0 + jnp.exp(-z))

    pltpu.sync_copy(out_v, out_hbm.at[pl.ds(base, BPW)])


@jax.jit
def _run(idx_t, idx_c, emb_t_T, emb_c_T, wb):
    mesh = plsc.VectorSubcoreMesh(core_axis_name="c", subcore_axis_name="s")
    gather = functools.partial(
        pl.kernel,
        mesh=mesh,
        compiler_params=pltpu.CompilerParams(needs_layout_passes=False),
        out_type=(jax.ShapeDtypeStruct((GROWS, 128), jnp.float32),
                  jax.ShapeDtypeStruct((GROWS, 128), jnp.float32)),
        scratch_types=[
            pltpu.VMEM((BATCH + L,), jnp.int32),
            pltpu.VMEM((BATCH + 2 * L,), jnp.int32),
            pltpu.VMEM((BATCH + 2 * L,), jnp.int32),
            pltpu.VMEM((EMB, 2 * SLAB_COLS), jnp.float32),
            pltpu.VMEM((NSLOTS * 128,), jnp.float32),
            pltpu.VMEM((2 * L,), jnp.int32),
            pltpu.VMEM((2 * L,), jnp.int32),
            pltpu.SemaphoreType.DMA,
            pltpu.SemaphoreType.DMA,
        ],
    )(_gather_body)
    g_t, g_c = gather(idx_t, idx_c, emb_t_T, emb_c_T)

    dot = functools.partial(
        pl.kernel,
        mesh=mesh,
        compiler_params=pltpu.CompilerParams(needs_layout_passes=False),
        out_type=jax.ShapeDtypeStruct((BATCH,), jnp.float32),
        scratch_types=[
            pltpu.VMEM((256, 128), jnp.float32),
            pltpu.VMEM((256, 128), jnp.float32),
            pltpu.VMEM((BPW,), jnp.float32),
            pltpu.VMEM((8, 128), jnp.float32),
            pltpu.VMEM((L * L,), jnp.float32),
            pltpu.SemaphoreType.DMA,
        ],
    )(_dot_body)
    return dot(g_t, g_c, wb)


def kernel(input_target, input_context, emb_target, emb_context, W, b):
    idx_t = input_target.reshape(-1).astype(jnp.int32)
    idx_c = input_context.reshape(-1).astype(jnp.int32)
    wb = jnp.concatenate([
        jnp.broadcast_to(W.reshape(1, 1), (1, 128)),
        jnp.broadcast_to(b.reshape(1, 1), (1, 128)),
        jnp.zeros((6, 128), jnp.float32),
    ], axis=0)
    out = _run(idx_t, idx_c, emb_target.T, emb_context.T, wb)
    return out.reshape(BATCH, 1)


# speedup vs baseline: 1.0641x; 1.0641x over previous
import functools

import jax
import jax.numpy as jnp
from jax import lax
from jax.experimental import pallas as pl
from jax.experimental.pallas import tpu as pltpu
from jax.experimental.pallas import tpu_sc as plsc

EMB = 64
BATCH = 16384
NC = 2
NS = 16
NW = NC * NS
BPW = BATCH // NW
L = 16
VOCAB = 1000000
FULL_TILES = VOCAB // 128
TAIL_C0 = FULL_TILES * 128
SLAB_TILES = 4
SLAB_COLS = SLAB_TILES * 128
SLAB_WORDS = EMB * SLAB_COLS
GROWS = BATCH + 2 * NW
NSLOTS = 32
SENTINEL = 1 << 30


SUPER = 8


def _gather_body(idx_t_hbm, idx_c_hbm, emb_t_hbm, emb_c_hbm,
                 g_t_hbm, g_c_hbm,
                 idx_all, pos_list, sub_pos, slab, rstage,
                 tmp_val, tmp_pos, sem_slab, sem_row):
    wid = lax.axis_index("s") * NC + lax.axis_index("c")
    lanes = lax.iota(jnp.int32, L)

    lo_tile = (wid * FULL_TILES) // NW
    hi_tile = ((wid + 1) * FULL_TILES) // NW
    nslabs = (hi_tile - lo_tile + SLAB_TILES - 1) // SLAB_TILES
    nsuper = (nslabs + SUPER - 1) // SUPER
    lo_col = lo_tile * 128
    hi_col = jnp.where(wid == NW - 1, VOCAB, hi_tile * 128)

    featv = [lanes + q * L for q in range(EMB // L)]

    def slab_c0(s):
        st = jnp.minimum(lo_tile + s * SLAB_TILES, hi_tile - SLAB_TILES)
        return st * 128

    def fire_slab(s, emb_hbm):
        halfc = lax.rem(s, 2) * SLAB_COLS
        c0 = slab_c0(s)
        for hr in range(4):
            pltpu.async_copy(
                emb_hbm.at[pl.ds(hr * 16, 16), pl.ds(c0, SLAB_COLS)],
                slab.at[pl.ds(hr * 16, 16), pl.ds(halfc, SLAB_COLS)],
                sem_slab)

    def drain_slab():
        pltpu.make_async_copy(
            emb_t_hbm.at[pl.ds(0, EMB), pl.ds(0, SLAB_COLS)],
            slab.at[pl.ds(0, EMB), pl.ds(0, SLAB_COLS)], sem_slab).wait()

    def drain_row():
        pltpu.make_async_copy(
            emb_t_hbm.at[0, pl.ds(0, 128)],
            rstage.at[pl.ds(0, 128)], sem_row).wait()

    def extract(list_ref, listlen, c0, c1, half, hh0, g_hbm):
        nv = (listlen + L - 1) // L

        def scan_one(kk, hh):
            posv = list_ref[pl.ds(kk * L, L)]
            valv = plsc.load_gather(idx_all, [posv])
            m = (valv >= c0) & (valv < c1)
            n = plsc.all_reduce_population_count(m)[0]
            plsc.store_compressed(tmp_val.at[pl.ds(0, L)], valv, mask=m)
            plsc.store_compressed(tmp_pos.at[pl.ds(0, L)], posv, mask=m)

            def hit_one(h, hh):
                cv = tmp_val[pl.ds(h, L)][0]
                pos = tmp_pos[pl.ds(h, L)][0]
                cl = half + (cv - c0)
                clv = jnp.full((L,), 0, jnp.int32) + cl
                slot = lax.rem(hh, NSLOTS)
                drain_row()
                for q in range(EMB // L):
                    colv = plsc.load_gather(slab, [featv[q], clv])
                    rstage[pl.ds(slot * 128 + q * L, L)] = colv
                pltpu.async_copy(rstage.at[pl.ds(slot * 128, 128)],
                                 g_hbm.at[pos], sem_row)
                return hh + 1

            return lax.fori_loop(0, n, hit_one, hh)

        return lax.fori_loop(0, nv, scan_one, hh0)

    for (idx_hbm, emb_hbm, g_hbm, padrow) in (
            (idx_t_hbm, emb_t_hbm, g_t_hbm, BATCH),
            (idx_c_hbm, emb_c_hbm, g_c_hbm, BATCH + NW)):
        pltpu.sync_copy(idx_hbm, idx_all.at[pl.ds(0, BATCH)])
        idx_all[pl.ds(BATCH, L)] = jnp.full((L,), SENTINEL, jnp.int32)

        fire_slab(0, emb_hbm)

        def build_one(k, count):
            ms, ns = [], []
            for u in range(4):
                v = idx_all[pl.ds((k * 4 + u) * L, L)]
                m = (v >= lo_col) & (v < hi_col)
                ms.append(m)
                ns.append(plsc.all_reduce_population_count(m)[0])
            for u in range(4):
                plsc.store_compressed(
                    pos_list.at[pl.ds(count, L)],
                    lanes + (k * 4 + u) * L, mask=ms[u])
                count = count + ns[u]
            return count

        count = lax.fori_loop(0, BATCH // L // 4, build_one, 0)
        for u in range(4):
            pos_list[pl.ds(count + u * L, L)] = jnp.full(
                (L,), BATCH, jnp.int32)

        for d in range(NSLOTS):
            pltpu.async_copy(rstage.at[pl.ds((d % NSLOTS) * 128, 128)],
                             g_hbm.at[padrow + d % NW], sem_row)

        def super_step(sp, hh):
            sc0 = slab_c0(sp * SUPER)
            sc1 = sc0 + SUPER * SLAB_COLS

            def filt(kk, scount):
                ps, ms, ns = [], [], []
                for u in range(4):
                    posv = pos_list[pl.ds((kk * 4 + u) * L, L)]
                    valv = plsc.load_gather(idx_all, [posv])
                    m = (valv >= sc0) & (valv < sc1)
                    ps.append(posv)
                    ms.append(m)
                    ns.append(plsc.all_reduce_population_count(m)[0])
                for u in range(4):
                    plsc.store_compressed(
                        sub_pos.at[pl.ds(scount, L)], ps[u], mask=ms[u])
                    scount = scount + ns[u]
                return scount

            scount = lax.fori_loop(0, (count + 4 * L - 1) // (4 * L),
                                   filt, 0)
            sub_pos[pl.ds(scount, L)] = jnp.full((L,), BATCH, jnp.int32)

            def slab_step(k, hh):
                sg = sp * SUPER + k
                fire_slab(sg + 1, emb_hbm)
                drain_slab()
                c0 = slab_c0(sg)
                hh = extract(sub_pos, scount, c0, c0 + SLAB_COLS,
                             lax.rem(sg, 2) * SLAB_COLS, hh, g_hbm)
                return hh

            return lax.fori_loop(0, SUPER, slab_step, hh)

        hh = lax.fori_loop(0, nsuper, super_step, 0)
        drain_slab()

        for f in range(EMB):
            pltpu.async_copy(
                emb_hbm.at[f, pl.ds(TAIL_C0, 64)],
                slab.at[f, pl.ds(0, 64)], sem_slab)
        pltpu.make_async_copy(
            emb_t_hbm.at[pl.ds(0, 8), pl.ds(0, SLAB_COLS)],
            slab.at[pl.ds(0, 8), pl.ds(0, SLAB_COLS)], sem_slab).wait()
        hh = extract(pos_list, count, TAIL_C0, TAIL_C0 + 64, 0, hh, g_hbm)

        def drain_rest(_, x):
            drain_row()
            return x

        lax.fori_loop(0, NSLOTS, drain_rest, 0)


def _dot_body(g_t_hbm, g_c_hbm, wb_hbm, out_hbm,
              t_loc, c_loc, out_v, wb_v, scr, sem):
    wid = lax.axis_index("s") * NC + lax.axis_index("c")
    base = wid * BPW
    lanes = lax.iota(jnp.int32, L)

    pltpu.sync_copy(wb_hbm, wb_v)

    HALF = 256
    for h in range(BPW // HALF):
        pltpu.sync_copy(
            g_t_hbm.at[pl.ds(base + h * HALF, HALF), pl.ds(0, 128)], t_loc)
        pltpu.sync_copy(
            g_c_hbm.at[pl.ds(base + h * HALF, HALF), pl.ds(0, 128)], c_loc)

        def group(g, _):
            for i in range(L):
                row = g * L + i
                s = t_loc[row, pl.ds(0, L)] * c_loc[row, pl.ds(0, L)]
                for q in range(1, EMB // L):
                    s = s + (t_loc[row, pl.ds(q * L, L)]
                             * c_loc[row, pl.ds(q * L, L)])
                scr[pl.ds(i * L, L)] = s
            acc = plsc.load_gather(scr, [lanes * L])
            for j in range(1, L):
                acc = acc + plsc.load_gather(scr, [lanes * L + j])
            out_v[pl.ds(h * HALF + g * L, L)] = acc
            return _

        lax.fori_loop(0, HALF // L, group, None)

    w = wb_v[0, pl.ds(0, L)]
    bb = wb_v[1, pl.ds(0, L)]
    for j in range(BPW // L):
        v = out_v[pl.ds(j * L, L)]
        z = v * w + bb
        out_v[pl.ds(j * L, L)] = 1.0 / (1.0 + jnp.exp(-z))

    pltpu.sync_copy(out_v, out_hbm.at[pl.ds(base, BPW)])


@jax.jit
def _run(idx_t, idx_c, emb_t_T, emb_c_T, wb):
    mesh = plsc.VectorSubcoreMesh(core_axis_name="c", subcore_axis_name="s")
    gather = functools.partial(
        pl.kernel,
        mesh=mesh,
        compiler_params=pltpu.CompilerParams(needs_layout_passes=False),
        out_type=(jax.ShapeDtypeStruct((GROWS, 128), jnp.float32),
                  jax.ShapeDtypeStruct((GROWS, 128), jnp.float32)),
        scratch_types=[
            pltpu.VMEM((BATCH + L,), jnp.int32),
            pltpu.VMEM((BATCH + 5 * L,), jnp.int32),
            pltpu.VMEM((BATCH + 2 * L,), jnp.int32),
            pltpu.VMEM((EMB, 2 * SLAB_COLS), jnp.float32),
            pltpu.VMEM((NSLOTS * 128,), jnp.float32),
            pltpu.VMEM((2 * L,), jnp.int32),
            pltpu.VMEM((2 * L,), jnp.int32),
            pltpu.SemaphoreType.DMA,
            pltpu.SemaphoreType.DMA,
        ],
    )(_gather_body)
    g_t, g_c = gather(idx_t, idx_c, emb_t_T, emb_c_T)

    dot = functools.partial(
        pl.kernel,
        mesh=mesh,
        compiler_params=pltpu.CompilerParams(needs_layout_passes=False),
        out_type=jax.ShapeDtypeStruct((BATCH,), jnp.float32),
        scratch_types=[
            pltpu.VMEM((256, 128), jnp.float32),
            pltpu.VMEM((256, 128), jnp.float32),
            pltpu.VMEM((BPW,), jnp.float32),
            pltpu.VMEM((8, 128), jnp.float32),
            pltpu.VMEM((L * L,), jnp.float32),
            pltpu.SemaphoreType.DMA,
        ],
    )(_dot_body)
    return dot(g_t, g_c, wb)


def kernel(input_target, input_context, emb_target, emb_context, W, b):
    idx_t = input_target.reshape(-1).astype(jnp.int32)
    idx_c = input_context.reshape(-1).astype(jnp.int32)
    wb = jnp.concatenate([
        jnp.broadcast_to(W.reshape(1, 1), (1, 128)),
        jnp.broadcast_to(b.reshape(1, 1), (1, 128)),
        jnp.zeros((6, 128), jnp.float32),
    ], axis=0)
    out = _run(idx_t, idx_c, emb_target.T, emb_context.T, wb)
    return out.reshape(BATCH, 1)
